# baseline (speedup 1.0000x reference)
import functools

import jax
import jax.numpy as jnp
from jax import lax
from jax.experimental import pallas as pl
from jax.experimental.pallas import tpu as pltpu
from jax.experimental.pallas import tpu_sc as plsc

_B = 4096
_D = 512
_K = 64
_E = 65536
_TEMP = 0.07
_PAIRS_PER_ANCHOR = _E // _B

_BM = 512
_NBLK = _B // _BM
_NBH = _NBLK // 2
_BH = _B // 2

_NC = 2
_NS = 16
_NW = _NC * _NS
_L = 16
_CH = 128

_NCHUNK = _B // 128


def _make_fused_body(half):
    is_final = half == 1

    def body(y_ref, adj_ref, a_all_ref, a_blk_ref,
             pd_in, cs_in, tr_in, dg_in, stat_in,
             dot_ref, stat_ref, spc_ref, pd_ref, cs_ref, tr_ref, dg_ref):
        i = pl.program_id(0)
        gi = i + half * _NBH
        x = y_ref[pl.ds(pl.multiple_of(gi * _BM, _BM), _BM), :]
        dot = lax.dot_general(
            x, y_ref[...], (((1,), (1,)), ((), ())),
            preferred_element_type=jnp.float32) * (1.0 / _TEMP)
        rowmax = jnp.max(dot, axis=1, keepdims=True)
        r = lax.broadcasted_iota(jnp.int32, dot.shape, 0) + gi * _BM
        cc = lax.broadcasted_iota(jnp.int32, dot.shape, 1)
        ex = jnp.where(r == cc, 0.0, jnp.exp(dot - rowmax))
        s = jnp.sum(ex, axis=1, keepdims=True)
        blocksum = jnp.sum(rowmax + jnp.log(s))
        prev = jnp.where(i == 0, stat_in[...], stat_ref[...])
        stat_ref[...] = prev + blocksum
        chunks = [dot[:, k * 128:(k + 1) * 128].reshape(1, _BM, 128)
                  for k in range(_NCHUNK)]
        dot_ref[...] = jnp.concatenate(chunks, axis=0)

        s_all = (a_all_ref[...] == lax.broadcasted_iota(
            jnp.int32, (_B, _K), 1)).astype(jnp.float32)
        s_blk = (a_blk_ref[...] == lax.broadcasted_iota(
            jnp.int32, (_BM, _K), 1)).astype(jnp.float32)
        p = lax.dot_general(
            adj_ref[...], s_all, (((1,), (0,)), ((), ())),
            preferred_element_type=jnp.float32)
        d_blk = jnp.sum(p, axis=1, keepdims=True)

        @pl.when(i == 0)
        def _():
            pd_ref[...] = pd_in[...]
            cs_ref[...] = cs_in[...]
            tr_ref[...] = tr_in[...]
            dg_ref[...] = dg_in[...]

        pd_ref[...] += jnp.sum(d_blk * s_blk, axis=0, keepdims=True)
        cs_ref[...] += jnp.sum(s_blk, axis=0, keepdims=True)
        tr_ref[...] += jnp.sum(p * s_blk)
        dg_ref[...] += jnp.sum(d_blk)

        @pl.when(i == _NBH - 1)
        def _():
            if is_final:
                m = jnp.sum(dg_ref[...]) * 0.5
                tr_pool = jnp.sum(tr_ref[...])
                pd = pd_ref[...]
                tr_norm = jnp.sum(pd * pd) / (2.0 * m)
                spectral = -(tr_pool - tr_norm) / (2.0 * m)
                cs = cs_ref[...]
                cs_norm = jnp.sqrt(jnp.sum(cs * cs))
                collapse = cs_norm / _B * jnp.sqrt(jnp.float32(_K)) - 1.0
                spc_ref[...] = jnp.full((1, 1), spectral + collapse,
                                        jnp.float32)
            else:
                spc_ref[...] = jnp.zeros((1, 1), jnp.float32)

    return body


def _make_fused_call(half):
    return pl.pallas_call(
        _make_fused_body(half),
        grid=(_NBH,),
        in_specs=[
            pl.BlockSpec((_B, _D), lambda i: (0, 0)),
            pl.BlockSpec((_BM, _B), lambda i, h=half: (i + h * _NBH, 0)),
            pl.BlockSpec((_B, 1), lambda i: (0, 0)),
            pl.BlockSpec((_BM, 1), lambda i, h=half: (i + h * _NBH, 0)),
            pl.BlockSpec((1, _K), lambda i: (0, 0)),
            pl.BlockSpec((1, _K), lambda i: (0, 0)),
            pl.BlockSpec((1, 1), lambda i: (0, 0)),
            pl.BlockSpec((1, 1), lambda i: (0, 0)),
            pl.BlockSpec((1, 1), lambda i: (0, 0)),
        ],
        out_specs=[
            pl.BlockSpec((_NCHUNK, _BM, 128), lambda i: (0, i, 0)),
            pl.BlockSpec((1, 1), lambda i: (0, 0)),
            pl.BlockSpec((1, 1), lambda i: (0, 0)),
            pl.BlockSpec((1, _K), lambda i: (0, 0)),
            pl.BlockSpec((1, _K), lambda i: (0, 0)),
            pl.BlockSpec((1, 1), lambda i: (0, 0)),
            pl.BlockSpec((1, 1), lambda i: (0, 0)),
        ],
        out_shape=[
            jax.ShapeDtypeStruct((_NCHUNK, _BH, 128), jnp.float32),
            jax.ShapeDtypeStruct((1, 1), jnp.float32),
            jax.ShapeDtypeStruct((1, 1), jnp.float32),
            jax.ShapeDtypeStruct((1, _K), jnp.float32),
            jax.ShapeDtypeStruct((1, _K), jnp.float32),
            jax.ShapeDtypeStruct((1, 1), jnp.float32),
            jax.ShapeDtypeStruct((1, 1), jnp.float32),
        ],
    )


_fused_calls = (_make_fused_call(0), _make_fused_call(1))


@functools.cache
def _make_pair_gather(half):
    npw = _E // 2 // _NW
    nch = npw // _CH
    e0 = half * (_E // 2)
    r0 = half * _BH

    @functools.partial(
        pl.kernel,
        out_type=jax.ShapeDtypeStruct((_NW * _L,), jnp.float32),
        mesh=plsc.VectorSubcoreMesh(core_axis_name="c", subcore_axis_name="s"),
        scratch_types=[
            pltpu.VMEM((npw,), jnp.int32),
            pltpu.VMEM((npw,), jnp.int32),
            pltpu.VMEM((nch, _CH), jnp.int32),
            pltpu.VMEM((nch, _CH), jnp.float32),
            pltpu.VMEM((_L,), jnp.float32),
            pltpu.SemaphoreType.DMA,
        ],
    )
    def pair_gather(dot_hbm, row_hbm, col_hbm, out_hbm,
                    row_v, col_v, idx_v, gat_v, acc_v, sem):
        wid = lax.axis_index("s") * _NC + lax.axis_index("c")
        base = e0 + wid * npw
        pltpu.sync_copy(row_hbm.at[pl.ds(base, npw)], row_v)
        pltpu.sync_copy(col_hbm.at[pl.ds(base, npw)], col_v)
        copies = []
        for j in range(nch):
            for t in range(_CH // _L):
                o = j * _CH + t * _L
                rv = row_v[pl.ds(o, _L)] - r0
                cv = col_v[pl.ds(o, _L)]
                idx_v[j, pl.ds(t * _L, _L)] = (
                    lax.shift_right_logical(cv, 7) * (_BH * 128)
                    + rv * 128 + (cv & 127))
            copies.append(
                pltpu.async_copy(dot_hbm.at[idx_v.at[j]], gat_v.at[j], sem))
        for cp in copies:
            cp.wait()
        acc = jnp.zeros((_L,), jnp.float32)
        for j in range(nch):
            for t in range(_CH // _L):
                acc = acc + gat_v[j, pl.ds(t * _L, _L)]
        acc_v[...] = acc
        pltpu.sync_copy(acc_v, out_hbm.at[pl.ds(wid * _L, _L)])

    return pair_gather


def kernel(out, row, col, val, assignment, adjacency):
    a2 = assignment.reshape(_B, 1)
    zk = jnp.zeros((1, _K), jnp.float32)
    z1 = jnp.zeros((1, 1), jnp.float32)
    dot0, stat0, _, pd0, cs0, tr0, dg0 = _fused_calls[0](
        out, adjacency, a2, a2, zk, zk, z1, z1, z1)
    dot1, stat1, spc, _, _, _, _ = _fused_calls[1](
        out, adjacency, a2, a2, pd0, cs0, tr0, dg0, stat0)
    p0 = _make_pair_gather(0)(dot0.reshape(_B * _BH), row, col)
    p1 = _make_pair_gather(1)(dot1.reshape(_B * _BH), row, col)
    pair_sum = jnp.sum(p0) + jnp.sum(p1)
    loss = ((_TEMP / _B) * stat1[0, 0]
            - (_TEMP / (_PAIRS_PER_ANCHOR * _B)) * pair_sum)
    return loss + spc[0, 0]
